# Initial kernel scaffold; baseline (speedup 1.0000x reference)
import jax
import jax.numpy as jnp
from jax.experimental import pallas as pl


def kernel(x, Wa, ba, Wp, bp, module_assign, batch):
    raise NotImplementedError("write your pallas kernel here")


# speedup vs baseline: 3.7486x; 3.7486x over previous
import jax
import jax.numpy as jnp
from jax.experimental import pallas as pl
from jax.experimental.pallas import tpu as pltpu

_NUM_MODULES = 11
_HIDDEN = 256
_B = 64
_NEG = -1e30


def _stage1_body(x_ref, w_ref, b_ref, m_ref, raw_ref, proj_ref, mmax_ref):
    i = pl.program_id(0)
    x = x_ref[...]
    w = w_ref[...]
    s = jnp.dot(x, w, preferred_element_type=jnp.float32,
                precision=jax.lax.Precision.HIGHEST) + b_ref[...]
    m = m_ref[...]
    iota = jax.lax.broadcasted_iota(jnp.int32, (1, _NUM_MODULES), 1)
    oh = (m == iota)
    sa = s[:, :_NUM_MODULES]
    sp = s[:, _NUM_MODULES:2 * _NUM_MODULES]
    raw_ref[...] = jnp.sum(jnp.where(oh, sa, 0.0), axis=1, keepdims=True)
    proj_ref[...] = jnp.sum(jnp.where(oh, sp, 0.0), axis=1, keepdims=True)
    tile_max = jnp.max(jnp.where(oh, sa, _NEG), axis=0, keepdims=True)

    @pl.when(i == 0)
    def _():
        mmax_ref[...] = jnp.full((1, _NUM_MODULES), _NEG, jnp.float32)

    mmax_ref[...] = jnp.maximum(mmax_ref[...], tile_max)


def _stage2_body(raw_ref, proj_ref, m_ref, c_ref, mmax_ref, out_ref,
                 num_acc, den_acc):
    i = pl.program_id(0)
    nt = pl.num_programs(0)
    raw = raw_ref[...]
    proj = proj_ref[...]
    m = m_ref[...]
    bt = c_ref[...]
    io_m = jax.lax.broadcasted_iota(jnp.int32, (1, _NUM_MODULES), 1)
    io_b = jax.lax.broadcasted_iota(jnp.int32, (1, _B), 1)
    ohm = (m == io_m).astype(jnp.float32)
    ohb = (bt == io_b).astype(jnp.float32)
    surr = mmax_ref[...]
    surr_n = jnp.sum(ohm * surr, axis=1, keepdims=True)
    ex = jnp.exp(raw - surr_n)
    mex = ohm * ex
    mw = mex * proj
    dims = (((0,), (0,)), ((), ()))
    dden = jax.lax.dot_general(ohb, mex, dims,
                               preferred_element_type=jnp.float32,
                               precision=jax.lax.Precision.HIGHEST)
    dnum = jax.lax.dot_general(ohb, mw, dims,
                               preferred_element_type=jnp.float32,
                               precision=jax.lax.Precision.HIGHEST)

    @pl.when(i == 0)
    def _():
        num_acc[...] = jnp.zeros((_B, _NUM_MODULES), jnp.float32)
        den_acc[...] = jnp.zeros((_B, _NUM_MODULES), jnp.float32)

    num_acc[...] += dnum
    den_acc[...] += dden

    @pl.when(i == nt - 1)
    def _():
        out_ref[...] = num_acc[...] / (den_acc[...] + 1e-16)


def kernel(x, Wa, ba, Wp, bp, module_assign, batch):
    n = x.shape[0]
    t1 = 2000
    t2 = 4000
    nt1 = n // t1
    nt2 = n // t2
    wcat = jnp.concatenate([Wa, Wp], axis=0).T
    bcat = jnp.concatenate([ba, bp], axis=0)[None, :]
    m_col = module_assign.astype(jnp.int32).reshape(n, 1)
    b_col = batch.astype(jnp.int32).reshape(n, 1)

    raw, proj, mmax = pl.pallas_call(
        _stage1_body,
        grid=(nt1,),
        in_specs=[
            pl.BlockSpec((t1, _HIDDEN), lambda i: (i, 0)),
            pl.BlockSpec((_HIDDEN, 2 * _NUM_MODULES), lambda i: (0, 0)),
            pl.BlockSpec((1, 2 * _NUM_MODULES), lambda i: (0, 0)),
            pl.BlockSpec((t1, 1), lambda i: (i, 0)),
        ],
        out_specs=[
            pl.BlockSpec((t1, 1), lambda i: (i, 0)),
            pl.BlockSpec((t1, 1), lambda i: (i, 0)),
            pl.BlockSpec((1, _NUM_MODULES), lambda i: (0, 0)),
        ],
        out_shape=[
            jax.ShapeDtypeStruct((n, 1), jnp.float32),
            jax.ShapeDtypeStruct((n, 1), jnp.float32),
            jax.ShapeDtypeStruct((1, _NUM_MODULES), jnp.float32),
        ],
    )(x, wcat, bcat, m_col)

    out = pl.pallas_call(
        _stage2_body,
        grid=(nt2,),
        in_specs=[
            pl.BlockSpec((t2, 1), lambda i: (i, 0)),
            pl.BlockSpec((t2, 1), lambda i: (i, 0)),
            pl.BlockSpec((t2, 1), lambda i: (i, 0)),
            pl.BlockSpec((t2, 1), lambda i: (i, 0)),
            pl.BlockSpec((1, _NUM_MODULES), lambda i: (0, 0)),
        ],
        out_specs=pl.BlockSpec((_B, _NUM_MODULES), lambda i: (0, 0)),
        out_shape=jax.ShapeDtypeStruct((_B, _NUM_MODULES), jnp.float32),
        scratch_shapes=[
            pltpu.VMEM((_B, _NUM_MODULES), jnp.float32),
            pltpu.VMEM((_B, _NUM_MODULES), jnp.float32),
        ],
    )(raw, proj, m_col, b_col, mmax)

    return out


# speedup vs baseline: 12.1049x; 3.2292x over previous
import jax
import jax.numpy as jnp
from jax.experimental import pallas as pl
from jax.experimental.pallas import tpu as pltpu

_NUM_MODULES = 11
_HIDDEN = 256
_B = 64
_NEG = -1e30


def _stage1_body(x_ref, w_ref, b_ref, m_ref, sps_ref, tmax_ref):
    x = x_ref[...]
    w = w_ref[...]
    s = jnp.dot(x, w, preferred_element_type=jnp.float32) + b_ref[...]
    m = m_ref[...]
    iota = jax.lax.broadcasted_iota(jnp.int32, (1, _NUM_MODULES), 1)
    oh = (m == iota)
    sa_m = jnp.where(oh, s[:, :_NUM_MODULES], _NEG)
    sps_ref[...] = jnp.concatenate(
        [sa_m, s[:, _NUM_MODULES:2 * _NUM_MODULES]], axis=1)
    tmax_ref[...] = jnp.max(sa_m, axis=0, keepdims=True)[None]


def _stage2_body(sps_ref, bt_ref, tmax_ref, out_ref, acc):
    i = pl.program_id(0)
    nt = pl.num_programs(0)
    sps = sps_ref[...]
    bt = bt_ref[0]
    mmax = jnp.max(tmax_ref[...], axis=0)
    surr = jnp.where(mmax < -1e29, 0.0, mmax)
    mex = jnp.exp(sps[:, :_NUM_MODULES] - surr)
    mw = mex * sps[:, _NUM_MODULES:2 * _NUM_MODULES]
    cat = jnp.concatenate([mex, mw], axis=1)
    io64 = jax.lax.broadcasted_iota(jnp.int32, (_B, sps.shape[0]), 0)
    ohbt = (io64 == bt).astype(jnp.float32)
    contrib = jnp.dot(ohbt, cat, preferred_element_type=jnp.float32)

    @pl.when(i == 0)
    def _():
        acc[...] = jnp.zeros((_B, 2 * _NUM_MODULES), jnp.float32)

    acc[...] += contrib

    @pl.when(i == nt - 1)
    def _():
        out_ref[...] = (acc[:, _NUM_MODULES:2 * _NUM_MODULES]
                        / (acc[:, :_NUM_MODULES] + 1e-16))


def kernel(x, Wa, ba, Wp, bp, module_assign, batch):
    n = x.shape[0]
    t1 = 4000
    t2 = 4000
    nt1 = n // t1
    nt2 = n // t2
    wcat = jnp.concatenate([Wa, Wp], axis=0).T
    bcat = jnp.concatenate([ba, bp], axis=0)[None, :]
    m_col = module_assign.astype(jnp.int32).reshape(n, 1)
    b_row = batch.astype(jnp.int32).reshape(nt2, 1, t2)

    sps, tmax = pl.pallas_call(
        _stage1_body,
        grid=(nt1,),
        in_specs=[
            pl.BlockSpec((t1, _HIDDEN), lambda i: (i, 0)),
            pl.BlockSpec((_HIDDEN, 2 * _NUM_MODULES), lambda i: (0, 0)),
            pl.BlockSpec((1, 2 * _NUM_MODULES), lambda i: (0, 0)),
            pl.BlockSpec((t1, 1), lambda i: (i, 0)),
        ],
        out_specs=[
            pl.BlockSpec((t1, 2 * _NUM_MODULES), lambda i: (i, 0)),
            pl.BlockSpec((1, 1, _NUM_MODULES), lambda i: (i, 0, 0)),
        ],
        out_shape=[
            jax.ShapeDtypeStruct((n, 2 * _NUM_MODULES), jnp.float32),
            jax.ShapeDtypeStruct((nt1, 1, _NUM_MODULES), jnp.float32),
        ],
        compiler_params=pltpu.CompilerParams(
            dimension_semantics=("parallel",)),
    )(x, wcat, bcat, m_col)

    out = pl.pallas_call(
        _stage2_body,
        grid=(nt2,),
        in_specs=[
            pl.BlockSpec((t2, 2 * _NUM_MODULES), lambda i: (i, 0)),
            pl.BlockSpec((1, 1, t2), lambda i: (i, 0, 0)),
            pl.BlockSpec((nt1, 1, _NUM_MODULES), lambda i: (0, 0, 0)),
        ],
        out_specs=pl.BlockSpec((_B, _NUM_MODULES), lambda i: (0, 0)),
        out_shape=jax.ShapeDtypeStruct((_B, _NUM_MODULES), jnp.float32),
        scratch_shapes=[
            pltpu.VMEM((_B, 2 * _NUM_MODULES), jnp.float32),
        ],
        compiler_params=pltpu.CompilerParams(
            dimension_semantics=("arbitrary",)),
    )(sps, b_row, tmax)

    return out


# speedup vs baseline: 12.8771x; 1.0638x over previous
import jax
import jax.numpy as jnp
from jax.experimental import pallas as pl
from jax.experimental.pallas import tpu as pltpu

_NUM_MODULES = 11
_HIDDEN = 256
_B = 64
_NEG = -1e30


def _stage1_body(x_ref, w_ref, b_ref, m_ref, sps_ref, tmax_ref):
    x = x_ref[...]
    w = w_ref[...]
    s = jnp.dot(x, w, preferred_element_type=jnp.float32) + b_ref[...]
    m = m_ref[...]
    iota = jax.lax.broadcasted_iota(jnp.int32, (1, _NUM_MODULES), 1)
    oh = (m == iota)
    sa_m = jnp.where(oh, s[:, :_NUM_MODULES], _NEG)
    sps_ref[...] = jnp.concatenate(
        [sa_m, s[:, _NUM_MODULES:2 * _NUM_MODULES]], axis=1)
    tmax_ref[...] = jnp.max(sa_m, axis=0, keepdims=True)[None]


def _stage2_body(sps_ref, bt_ref, tmax_ref, out_ref, acc):
    i = pl.program_id(0)
    nt = pl.num_programs(0)
    sps = sps_ref[...]
    bt = bt_ref[0]
    mmax = jnp.max(tmax_ref[...], axis=0)
    surr = jnp.where(mmax < -1e29, 0.0, mmax)
    mex = jnp.exp(sps[:, :_NUM_MODULES] - surr)
    mw = mex * sps[:, _NUM_MODULES:2 * _NUM_MODULES]
    cat = jnp.concatenate([mex, mw], axis=1)
    io64 = jax.lax.broadcasted_iota(jnp.int32, (_B, sps.shape[0]), 0)
    ohbt = (io64 == bt).astype(jnp.float32)
    contrib = jnp.dot(ohbt, cat, preferred_element_type=jnp.float32)

    @pl.when(i == 0)
    def _():
        acc[...] = jnp.zeros((_B, 2 * _NUM_MODULES), jnp.float32)

    acc[...] += contrib

    @pl.when(i == nt - 1)
    def _():
        out_ref[...] = (acc[:, _NUM_MODULES:2 * _NUM_MODULES]
                        / (acc[:, :_NUM_MODULES] + 1e-16))


def kernel(x, Wa, ba, Wp, bp, module_assign, batch):
    n = x.shape[0]
    t1 = 10000
    t2 = 10000
    nt1 = n // t1
    nt2 = n // t2
    wcat = jnp.concatenate([Wa, Wp], axis=0).T
    bcat = jnp.concatenate([ba, bp], axis=0)[None, :]
    m_col = module_assign.astype(jnp.int32).reshape(n, 1)
    b_row = batch.astype(jnp.int32).reshape(nt2, 1, t2)

    sps, tmax = pl.pallas_call(
        _stage1_body,
        grid=(nt1,),
        in_specs=[
            pl.BlockSpec((t1, _HIDDEN), lambda i: (i, 0)),
            pl.BlockSpec((_HIDDEN, 2 * _NUM_MODULES), lambda i: (0, 0)),
            pl.BlockSpec((1, 2 * _NUM_MODULES), lambda i: (0, 0)),
            pl.BlockSpec((t1, 1), lambda i: (i, 0)),
        ],
        out_specs=[
            pl.BlockSpec((t1, 2 * _NUM_MODULES), lambda i: (i, 0)),
            pl.BlockSpec((1, 1, _NUM_MODULES), lambda i: (i, 0, 0)),
        ],
        out_shape=[
            jax.ShapeDtypeStruct((n, 2 * _NUM_MODULES), jnp.float32),
            jax.ShapeDtypeStruct((nt1, 1, _NUM_MODULES), jnp.float32),
        ],
        compiler_params=pltpu.CompilerParams(
            dimension_semantics=("parallel",)),
    )(x, wcat, bcat, m_col)

    out = pl.pallas_call(
        _stage2_body,
        grid=(nt2,),
        in_specs=[
            pl.BlockSpec((t2, 2 * _NUM_MODULES), lambda i: (i, 0)),
            pl.BlockSpec((1, 1, t2), lambda i: (i, 0, 0)),
            pl.BlockSpec((nt1, 1, _NUM_MODULES), lambda i: (0, 0, 0)),
        ],
        out_specs=pl.BlockSpec((_B, _NUM_MODULES), lambda i: (0, 0)),
        out_shape=jax.ShapeDtypeStruct((_B, _NUM_MODULES), jnp.float32),
        scratch_shapes=[
            pltpu.VMEM((_B, 2 * _NUM_MODULES), jnp.float32),
        ],
        compiler_params=pltpu.CompilerParams(
            dimension_semantics=("arbitrary",)),
    )(sps, b_row, tmax)

    return out


# speedup vs baseline: 17.1740x; 1.3337x over previous
import jax
import jax.numpy as jnp
from jax.experimental import pallas as pl
from jax.experimental.pallas import tpu as pltpu

_NUM_MODULES = 11
_HIDDEN = 256
_B = 64
_NEG = -1e30


def _fused_body(x_ref, w_ref, b_ref, m_ref, bt_ref, out_ref, acc, runmax):
    i = pl.program_id(0)
    nt = pl.num_programs(0)
    x = x_ref[...]
    s = jnp.dot(x, w_ref[...], preferred_element_type=jnp.float32) + b_ref[...]
    m = m_ref[...]
    iota = jax.lax.broadcasted_iota(jnp.int32, (1, _NUM_MODULES), 1)
    oh = (m == iota)
    sa = jnp.where(oh, s[:, :_NUM_MODULES], _NEG)
    tmax = jnp.max(sa, axis=0, keepdims=True)

    @pl.when(i == 0)
    def _():
        acc[...] = jnp.zeros((_B, 2 * _NUM_MODULES), jnp.float32)
        runmax[...] = jnp.full((1, _NUM_MODULES), _NEG, jnp.float32)

    old_raw = runmax[...]
    new_raw = jnp.maximum(old_raw, tmax)
    runmax[...] = new_raw
    stab_old = jnp.where(old_raw < -1e29, 0.0, old_raw)
    stab_new = jnp.where(new_raw < -1e29, 0.0, new_raw)
    factor = jnp.exp(stab_old - stab_new)

    mex = jnp.exp(sa - stab_new)
    mw = mex * s[:, _NUM_MODULES:2 * _NUM_MODULES]
    cat = jnp.concatenate([mex, mw], axis=1)
    bt = bt_ref[0]
    io64 = jax.lax.broadcasted_iota(jnp.int32, (_B, x.shape[0]), 0)
    ohbt = (io64 == bt).astype(jnp.float32)
    contrib = jnp.dot(ohbt, cat, preferred_element_type=jnp.float32)

    facc = jnp.concatenate([factor, factor], axis=1)
    acc[...] = acc[...] * facc + contrib

    @pl.when(i == nt - 1)
    def _():
        out_ref[...] = (acc[:, _NUM_MODULES:2 * _NUM_MODULES]
                        / (acc[:, :_NUM_MODULES] + 1e-16))


def kernel(x, Wa, ba, Wp, bp, module_assign, batch):
    n = x.shape[0]
    t = 10000
    nt = n // t
    wcat = jnp.concatenate([Wa, Wp], axis=0).T
    bcat = jnp.concatenate([ba, bp], axis=0)[None, :]
    m_col = module_assign.astype(jnp.int32).reshape(n, 1)
    b_row = batch.astype(jnp.int32).reshape(nt, 1, t)

    out = pl.pallas_call(
        _fused_body,
        grid=(nt,),
        in_specs=[
            pl.BlockSpec((t, _HIDDEN), lambda i: (i, 0)),
            pl.BlockSpec((_HIDDEN, 2 * _NUM_MODULES), lambda i: (0, 0)),
            pl.BlockSpec((1, 2 * _NUM_MODULES), lambda i: (0, 0)),
            pl.BlockSpec((t, 1), lambda i: (i, 0)),
            pl.BlockSpec((1, 1, t), lambda i: (i, 0, 0)),
        ],
        out_specs=pl.BlockSpec((_B, _NUM_MODULES), lambda i: (0, 0)),
        out_shape=jax.ShapeDtypeStruct((_B, _NUM_MODULES), jnp.float32),
        scratch_shapes=[
            pltpu.VMEM((_B, 2 * _NUM_MODULES), jnp.float32),
            pltpu.VMEM((1, _NUM_MODULES), jnp.float32),
        ],
        compiler_params=pltpu.CompilerParams(
            dimension_semantics=("arbitrary",)),
    )(x, wcat, bcat, m_col, b_row)

    return out
